# baseline (speedup 1.0000x reference)
import functools

import jax
import jax.numpy as jnp
from jax import lax
from jax.experimental import pallas as pl
from jax.experimental.pallas import tpu as pltpu
from jax.experimental.pallas import tpu_sc as plsc

_N_ROWS = 4096 * 200
_D = 128
_CB = 8
_NCOMBO = 3 ** _CB
_N_GROUPS = _N_ROWS // _CB
_NC, _NS = 2, 16
_NW = _NC * _NS
_G_PER_W = _N_GROUPS // _NW
_CHUNK = 32
_NIT = _G_PER_W // _CHUNK
_CBLK = 729
_IBLK = 80


def _combo_body(table_ref, out_ref):
    c0 = pl.program_id(0) * _CBLK
    c = c0 + lax.broadcasted_iota(jnp.int32, (_CBLK, 1), 0)
    t0 = table_ref[0, :].reshape(1, _D)
    t1 = table_ref[1, :].reshape(1, _D)
    t2 = table_ref[2, :].reshape(1, _D)
    for j in range(_CB):
        dig = (c // (3 ** (_CB - 1 - j))) % 3
        out_ref[:, j, :] = jnp.where(dig == 0, t0, jnp.where(dig == 1, t1, t2))


_build_combo = pl.pallas_call(
    _combo_body,
    grid=(_NCOMBO // _CBLK,),
    in_specs=[pl.BlockSpec((3, _D), lambda i: (0, 0))],
    out_specs=pl.BlockSpec((_CBLK, _CB, _D), lambda i: (i, 0, 0)),
    out_shape=jax.ShapeDtypeStruct((_NCOMBO, _CB, _D), jnp.float32),
)


def _combine_body(x_ref, out_ref):
    c = x_ref[:, :, 0]
    for j in range(1, _CB):
        c = c * 3 + x_ref[:, :, j]
    out_ref[...] = c


_combine_idx = pl.pallas_call(
    _combine_body,
    grid=(_N_GROUPS // 128 // _IBLK,),
    in_specs=[pl.BlockSpec((_IBLK, 128, _CB), lambda i: (i, 0, 0))],
    out_specs=pl.BlockSpec((_IBLK, 128), lambda i: (i, 0)),
    out_shape=jax.ShapeDtypeStruct((_N_GROUPS // 128, 128), jnp.int32),
)


_mesh = plsc.VectorSubcoreMesh(core_axis_name="c", subcore_axis_name="s")


@functools.partial(
    pl.kernel,
    mesh=_mesh,
    out_type=jax.ShapeDtypeStruct((_N_GROUPS, _CB * _D), jnp.float32),
    scratch_types=[
        pltpu.VMEM((_G_PER_W,), jnp.int32),
        pltpu.VMEM((2, _CHUNK, _CB * _D), jnp.float32),
        pltpu.SemaphoreType.DMA,
        pltpu.SemaphoreType.DMA,
        pltpu.SemaphoreType.DMA,
        pltpu.SemaphoreType.DMA,
    ],
)
def _gather_rows(idx_hbm, combo_hbm, out_hbm, idx_v, rows_v, gs0, gs1, ss0, ss1):
    wid = lax.axis_index("s") * _NC + lax.axis_index("c")
    base = wid * _G_PER_W
    pltpu.sync_copy(idx_hbm.at[pl.ds(base, _G_PER_W)], idx_v)

    gsem = (gs0, gs1)
    ssem = (ss0, ss1)

    def fire_gather(it, b):
        pltpu.async_copy(
            combo_hbm.at[idx_v.at[pl.ds(it * _CHUNK, _CHUNK)]],
            rows_v.at[b],
            gsem[b],
        )

    def wait_gather(b):
        pltpu.make_async_copy(
            out_hbm.at[pl.ds(0, _CHUNK)], rows_v.at[b], gsem[b]
        ).wait()

    def fire_store(it, b):
        pltpu.async_copy(
            rows_v.at[b], out_hbm.at[pl.ds(base + it * _CHUNK, _CHUNK)], ssem[b]
        )

    def wait_store(b):
        pltpu.make_async_copy(
            rows_v.at[b], out_hbm.at[pl.ds(0, _CHUNK)], ssem[b]
        ).wait()

    fire_gather(0, 0)
    wait_gather(0)
    fire_store(0, 0)
    fire_gather(1, 1)
    wait_gather(1)
    fire_store(1, 1)

    def pair(p, _):
        it0 = 2 * p
        wait_store(0)
        fire_gather(it0, 0)
        wait_gather(0)
        fire_store(it0, 0)
        wait_store(1)
        fire_gather(it0 + 1, 1)
        wait_gather(1)
        fire_store(it0 + 1, 1)
        return ()

    lax.fori_loop(1, _NIT // 2, pair, ())
    wait_store(0)
    wait_store(1)


def kernel(x, table):
    x3 = x.reshape(_N_GROUPS // 128, 128, _CB).astype(jnp.int32)
    cidx = _combine_idx(x3).reshape(_N_GROUPS)
    combo = _build_combo(table).reshape(_NCOMBO, _CB * _D)
    out = _gather_rows(cidx, combo)
    return out.reshape(x.shape[0], x.shape[1], _D)


# speedup vs baseline: 2.7660x; 2.7660x over previous
import functools

import jax
import jax.numpy as jnp
from jax import lax
from jax.experimental import pallas as pl
from jax.experimental.pallas import tpu as pltpu
from jax.experimental.pallas import tpu_sc as plsc

_N_ROWS = 4096 * 200
_D = 128
_NC, _NS = 2, 16
_NW = _NC * _NS
_ROWS_PER_W = _N_ROWS // _NW
_CHUNK = 128
_NIT = _ROWS_PER_W // _CHUNK
_NB = 4
_K = 4096
_NSL = _ROWS_PER_W // 16
_GRP = _K // 16


_mesh = plsc.VectorSubcoreMesh(core_axis_name="c", subcore_axis_name="s")


@functools.partial(
    pl.kernel,
    mesh=_mesh,
    out_type=jax.ShapeDtypeStruct((_N_ROWS, _D), jnp.float32),
    scratch_types=[
        pltpu.VMEM((_ROWS_PER_W,), jnp.int32),
        pltpu.VMEM((_NB, _CHUNK, _D), jnp.float32),
        pltpu.SemaphoreType.DMA,
        pltpu.SemaphoreType.DMA,
        pltpu.SemaphoreType.DMA,
        pltpu.SemaphoreType.DMA,
        pltpu.SemaphoreType.DMA,
        pltpu.SemaphoreType.DMA,
        pltpu.SemaphoreType.DMA,
        pltpu.SemaphoreType.DMA,
    ],
)
def _gather_rows(idx_hbm, table_hbm, out_hbm, idx_v, rows_v, *sems):
    gsem = sems[:_NB]
    ssem = sems[_NB:]
    wid = lax.axis_index("s") * _NC + lax.axis_index("c")
    base = wid * _ROWS_PER_W
    pltpu.sync_copy(idx_hbm.at[pl.ds(base, _ROWS_PER_W)], idx_v)

    lane_off = 3 * lax.iota(jnp.int32, 16)
    phase = wid * (_GRP // _NW)

    def spread(s8, _):
        for u in range(8):
            s = s8 * 8 + u
            rep = jnp.full(
                (16,), 3 * 16 * lax.rem(s + phase, _GRP), dtype=jnp.int32
            )
            sl = pl.ds(s * 16, 16)
            idx_v[sl] = idx_v[sl] + lane_off + rep
        return ()

    lax.fori_loop(0, _NSL // 8, spread, ())

    def fire_gather(it, b):
        pltpu.async_copy(
            table_hbm.at[idx_v.at[pl.ds(it * _CHUNK, _CHUNK)]],
            rows_v.at[b],
            gsem[b],
        )

    def wait_gather(b):
        pltpu.make_async_copy(
            out_hbm.at[pl.ds(0, _CHUNK)], rows_v.at[b], gsem[b]
        ).wait()

    def fire_store(it, b):
        pltpu.async_copy(
            rows_v.at[b], out_hbm.at[pl.ds(base + it * _CHUNK, _CHUNK)], ssem[b]
        )

    def wait_store(b):
        pltpu.make_async_copy(
            rows_v.at[b], out_hbm.at[pl.ds(0, _CHUNK)], ssem[b]
        ).wait()

    fire_gather(0, 0)
    for it in range(_NB):
        b, b1 = it % _NB, (it + 1) % _NB
        if it + 1 - _NB >= 0:
            wait_store(b1)
        fire_gather(it + 1, b1)
        wait_gather(b)
        fire_store(it, b)

    def quad(q, _):
        for u in range(_NB):
            it = q * _NB + u
            b, b1 = u, (u + 1) % _NB
            wait_store(b1)
            fire_gather(it + 1, b1)
            wait_gather(b)
            fire_store(it, b)
        return ()

    lax.fori_loop(1, _NIT // _NB - 1, quad, ())

    for u in range(_NB):
        it = _NIT - _NB + u
        b, b1 = u, (u + 1) % _NB
        if it < _NIT - 1:
            wait_store(b1)
            fire_gather(it + 1, b1)
        wait_gather(b)
        fire_store(it, b)
    for b in range(_NB):
        wait_store(b)


def kernel(x, table):
    idx = x.reshape(_N_ROWS).astype(jnp.int32)
    rep_table = jnp.tile(table, (_K, 1))
    out = _gather_rows(idx, rep_table)
    return out.reshape(x.shape[0], x.shape[1], _D)


# speedup vs baseline: 4.9753x; 1.7987x over previous
import functools

import jax
import jax.numpy as jnp
from jax import lax
from jax.experimental import pallas as pl
from jax.experimental.pallas import tpu as pltpu
from jax.experimental.pallas import tpu_sc as plsc

_N_ROWS = 4096 * 200
_D = 128
_NC, _NS = 2, 16
_NW = _NC * _NS
_ROWS_PER_W = _N_ROWS // _NW
_CHUNK = 128
_NIT = _ROWS_PER_W // _CHUNK
_NB = 4
_K = 1024
_NSL = _ROWS_PER_W // 16
_GRP = _K // 16
_TROWS = 3 * _K
_STG = _TROWS // _NS


_mesh = plsc.VectorSubcoreMesh(core_axis_name="c", subcore_axis_name="s")


@functools.partial(
    pl.kernel,
    mesh=_mesh,
    out_type=jax.ShapeDtypeStruct((_N_ROWS, _D), jnp.float32),
    scratch_types=[
        pltpu.VMEM((_ROWS_PER_W,), jnp.int32),
        pltpu.VMEM((_NB, _CHUNK, _D), jnp.float32),
        pltpu.VMEM_SHARED((_TROWS, _D), jnp.float32),
        pltpu.SemaphoreType.DMA,
        pltpu.SemaphoreType.DMA,
        pltpu.SemaphoreType.DMA,
        pltpu.SemaphoreType.DMA,
        pltpu.SemaphoreType.DMA,
        pltpu.SemaphoreType.DMA,
        pltpu.SemaphoreType.DMA,
        pltpu.SemaphoreType.DMA,
    ],
)
def _gather_rows(idx_hbm, table_hbm, out_hbm, idx_v, rows_v, table_sh, *sems):
    gsem = sems[:_NB]
    ssem = sems[_NB:]
    sid = lax.axis_index("s")
    wid = sid * _NC + lax.axis_index("c")
    base = wid * _ROWS_PER_W
    pltpu.sync_copy(
        table_hbm.at[pl.ds(sid * _STG, _STG)],
        table_sh.at[pl.ds(sid * _STG, _STG)],
    )
    pltpu.sync_copy(idx_hbm.at[pl.ds(base, _ROWS_PER_W)], idx_v)

    lane_off = 3 * lax.iota(jnp.int32, 16)
    phase = wid * (_GRP // _NW)

    def spread(s8, _):
        for u in range(8):
            s = s8 * 8 + u
            rep = jnp.full(
                (16,), 3 * 16 * lax.rem(s + phase, _GRP), dtype=jnp.int32
            )
            sl = pl.ds(s * 16, 16)
            idx_v[sl] = idx_v[sl] + lane_off + rep
        return ()

    lax.fori_loop(0, _NSL // 8, spread, ())
    plsc.subcore_barrier()

    def fire_gather(it, b):
        pltpu.async_copy(
            table_sh.at[idx_v.at[pl.ds(it * _CHUNK, _CHUNK)]],
            rows_v.at[b],
            gsem[b],
        )

    def wait_gather(b):
        pltpu.make_async_copy(
            out_hbm.at[pl.ds(0, _CHUNK)], rows_v.at[b], gsem[b]
        ).wait()

    def fire_store(it, b):
        pltpu.async_copy(
            rows_v.at[b], out_hbm.at[pl.ds(base + it * _CHUNK, _CHUNK)], ssem[b]
        )

    def wait_store(b):
        pltpu.make_async_copy(
            rows_v.at[b], out_hbm.at[pl.ds(0, _CHUNK)], ssem[b]
        ).wait()

    fire_gather(0, 0)
    for it in range(_NB):
        b, b1 = it % _NB, (it + 1) % _NB
        if it + 1 - _NB >= 0:
            wait_store(b1)
        fire_gather(it + 1, b1)
        wait_gather(b)
        fire_store(it, b)

    def quad(q, _):
        for u in range(_NB):
            it = q * _NB + u
            b, b1 = u, (u + 1) % _NB
            wait_store(b1)
            fire_gather(it + 1, b1)
            wait_gather(b)
            fire_store(it, b)
        return ()

    lax.fori_loop(1, _NIT // _NB - 1, quad, ())

    for u in range(_NB):
        it = _NIT - _NB + u
        b, b1 = u, (u + 1) % _NB
        if it < _NIT - 1:
            wait_store(b1)
            fire_gather(it + 1, b1)
        wait_gather(b)
        fire_store(it, b)
    for b in range(_NB):
        wait_store(b)


def kernel(x, table):
    idx = x.reshape(_N_ROWS).astype(jnp.int32)
    rep_table = jnp.tile(table, (_K, 1))
    out = _gather_rows(idx, rep_table)
    return out.reshape(x.shape[0], x.shape[1], _D)


# speedup vs baseline: 5.0583x; 1.0167x over previous
import functools

import jax
import jax.numpy as jnp
from jax import lax
from jax.experimental import pallas as pl
from jax.experimental.pallas import tpu as pltpu
from jax.experimental.pallas import tpu_sc as plsc

_N_ROWS = 4096 * 200
_D = 128
_NC, _NS = 2, 16
_NW = _NC * _NS
_ROWS_PER_W = _N_ROWS // _NW
_CHUNK = 128
_NIT = _ROWS_PER_W // _CHUNK
_NB = 4
_K = 128
_NSL = _ROWS_PER_W // 16
_GRP = _K // 16
_TROWS = 3 * _K
_STG = _TROWS // _NS


_mesh = plsc.VectorSubcoreMesh(core_axis_name="c", subcore_axis_name="s")


@functools.partial(
    pl.kernel,
    mesh=_mesh,
    out_type=jax.ShapeDtypeStruct((_N_ROWS, _D), jnp.float32),
    scratch_types=[
        pltpu.VMEM((_ROWS_PER_W,), jnp.int32),
        pltpu.VMEM((_NB, _CHUNK, _D), jnp.float32),
        pltpu.VMEM_SHARED((_TROWS, _D), jnp.float32),
        pltpu.SemaphoreType.DMA,
        pltpu.SemaphoreType.DMA,
        pltpu.SemaphoreType.DMA,
        pltpu.SemaphoreType.DMA,
        pltpu.SemaphoreType.DMA,
        pltpu.SemaphoreType.DMA,
        pltpu.SemaphoreType.DMA,
        pltpu.SemaphoreType.DMA,
    ],
)
def _gather_rows(idx_hbm, table_hbm, out_hbm, idx_v, rows_v, table_sh, *sems):
    gsem = sems[:_NB]
    ssem = sems[_NB:]
    sid = lax.axis_index("s")
    wid = sid * _NC + lax.axis_index("c")
    base = wid * _ROWS_PER_W
    pltpu.sync_copy(
        table_hbm.at[pl.ds(sid * _STG, _STG)],
        table_sh.at[pl.ds(sid * _STG, _STG)],
    )
    pltpu.sync_copy(idx_hbm.at[pl.ds(base, _ROWS_PER_W)], idx_v)

    lane_off = 3 * lax.iota(jnp.int32, 16)

    def spread(s8, _):
        for u in range(8):
            s = s8 * 8 + u
            sl = pl.ds(s * 16, 16)
            idx_v[sl] = idx_v[sl] + lane_off
        return ()

    lax.fori_loop(0, _NSL // 8, spread, ())
    plsc.subcore_barrier()

    def fire_gather(it, b):
        pltpu.async_copy(
            table_sh.at[idx_v.at[pl.ds(it * _CHUNK, _CHUNK)]],
            rows_v.at[b],
            gsem[b],
        )

    def wait_gather(b):
        pltpu.make_async_copy(
            out_hbm.at[pl.ds(0, _CHUNK)], rows_v.at[b], gsem[b]
        ).wait()

    def fire_store(it, b):
        pltpu.async_copy(
            rows_v.at[b], out_hbm.at[pl.ds(base + it * _CHUNK, _CHUNK)], ssem[b]
        )

    def wait_store(b):
        pltpu.make_async_copy(
            rows_v.at[b], out_hbm.at[pl.ds(0, _CHUNK)], ssem[b]
        ).wait()

    fire_gather(0, 0)
    for it in range(_NB):
        b, b1 = it % _NB, (it + 1) % _NB
        if it + 1 - _NB >= 0:
            wait_store(b1)
        fire_gather(it + 1, b1)
        wait_gather(b)
        fire_store(it, b)

    def quad(q, _):
        for u in range(_NB):
            it = q * _NB + u
            b, b1 = u, (u + 1) % _NB
            wait_store(b1)
            fire_gather(it + 1, b1)
            wait_gather(b)
            fire_store(it, b)
        return ()

    lax.fori_loop(1, _NIT // _NB - 1, quad, ())

    for u in range(_NB):
        it = _NIT - _NB + u
        b, b1 = u, (u + 1) % _NB
        if it < _NIT - 1:
            wait_store(b1)
            fire_gather(it + 1, b1)
        wait_gather(b)
        fire_store(it, b)
    for b in range(_NB):
        wait_store(b)


def kernel(x, table):
    idx = x.reshape(_N_ROWS).astype(jnp.int32)
    rep_table = jnp.tile(table, (_K, 1))
    out = _gather_rows(idx, rep_table)
    return out.reshape(x.shape[0], x.shape[1], _D)


# speedup vs baseline: 5.0643x; 1.0012x over previous
import functools

import jax
import jax.numpy as jnp
from jax import lax
from jax.experimental import pallas as pl
from jax.experimental.pallas import tpu as pltpu
from jax.experimental.pallas import tpu_sc as plsc

_N_ROWS = 4096 * 200
_D = 128
_NC, _NS = 2, 16
_NW = _NC * _NS
_ROWS_PER_W = _N_ROWS // _NW
_CHUNK = 128
_NIT = _ROWS_PER_W // _CHUNK
_NB = 4
_K = 128
_NSL = _ROWS_PER_W // 16
_GRP = _K // 16
_TROWS = 3 * _K
_STG = _TROWS // _NS


_mesh = plsc.VectorSubcoreMesh(core_axis_name="c", subcore_axis_name="s")


@functools.partial(
    pl.kernel,
    mesh=_mesh,
    out_type=jax.ShapeDtypeStruct((_N_ROWS, _D), jnp.float32),
    scratch_types=[
        pltpu.VMEM((_ROWS_PER_W,), jnp.int32),
        pltpu.VMEM((_NB, _CHUNK, _D), jnp.float32),
        pltpu.VMEM_SHARED((_TROWS, _D), jnp.float32),
        pltpu.SemaphoreType.DMA,
        pltpu.SemaphoreType.DMA,
        pltpu.SemaphoreType.DMA,
        pltpu.SemaphoreType.DMA,
        pltpu.SemaphoreType.DMA,
        pltpu.SemaphoreType.DMA,
        pltpu.SemaphoreType.DMA,
        pltpu.SemaphoreType.DMA,
    ],
)
def _gather_rows(idx_hbm, table_hbm, out_hbm, idx_v, rows_v, table_sh, *sems):
    gsem = sems[:_NB]
    ssem = sems[_NB:]
    sid = lax.axis_index("s")
    wid = sid * _NC + lax.axis_index("c")
    base = wid * _ROWS_PER_W
    pltpu.sync_copy(
        table_hbm.at[pl.ds(sid * _STG, _STG)],
        table_sh.at[pl.ds(sid * _STG, _STG)],
    )
    pltpu.sync_copy(idx_hbm.at[pl.ds(base, _ROWS_PER_W)], idx_v)

    plsc.subcore_barrier()

    lane_off = 3 * lax.iota(jnp.int32, 16)

    def fire_gather(it, b):
        for u in range(_CHUNK // 16):
            sl = pl.ds(it * _CHUNK + u * 16, 16)
            idx_v[sl] = idx_v[sl] + lane_off
        pltpu.async_copy(
            table_sh.at[idx_v.at[pl.ds(it * _CHUNK, _CHUNK)]],
            rows_v.at[b],
            gsem[b],
        )

    def wait_gather(b):
        pltpu.make_async_copy(
            out_hbm.at[pl.ds(0, _CHUNK)], rows_v.at[b], gsem[b]
        ).wait()

    def fire_store(it, b):
        pltpu.async_copy(
            rows_v.at[b], out_hbm.at[pl.ds(base + it * _CHUNK, _CHUNK)], ssem[b]
        )

    def wait_store(b):
        pltpu.make_async_copy(
            rows_v.at[b], out_hbm.at[pl.ds(0, _CHUNK)], ssem[b]
        ).wait()

    fire_gather(0, 0)
    for it in range(_NB):
        b, b1 = it % _NB, (it + 1) % _NB
        if it + 1 - _NB >= 0:
            wait_store(b1)
        fire_gather(it + 1, b1)
        wait_gather(b)
        fire_store(it, b)

    def quad(q, _):
        for u in range(_NB):
            it = q * _NB + u
            b, b1 = u, (u + 1) % _NB
            wait_store(b1)
            fire_gather(it + 1, b1)
            wait_gather(b)
            fire_store(it, b)
        return ()

    lax.fori_loop(1, _NIT // _NB - 1, quad, ())

    for u in range(_NB):
        it = _NIT - _NB + u
        b, b1 = u, (u + 1) % _NB
        if it < _NIT - 1:
            wait_store(b1)
            fire_gather(it + 1, b1)
        wait_gather(b)
        fire_store(it, b)
    for b in range(_NB):
        wait_store(b)


def kernel(x, table):
    idx = x.reshape(_N_ROWS).astype(jnp.int32)
    rep_table = jnp.tile(table, (_K, 1))
    out = _gather_rows(idx, rep_table)
    return out.reshape(x.shape[0], x.shape[1], _D)
